# baseline (speedup 1.0000x reference)
import functools

import jax
import jax.numpy as jnp
from jax import lax
from jax.experimental import pallas as pl
from jax.experimental.pallas import tpu as pltpu
from jax.experimental.pallas import tpu_sc as plsc


@functools.cache
def _make_sc_gather(V, D, S, B, NT):
    info = plsc.get_sparse_core_info()
    NC, NS = info.num_cores, info.num_subcores
    NW = NC * NS
    NB = NW // 2
    assert B % NB == 0 and S % 2 == 0
    BW = B // NB
    SH = S // 2
    DPT = 128 // D
    mesh = plsc.VectorSubcoreMesh(core_axis_name="c", subcore_axis_name="s")

    @functools.partial(
        pl.kernel,
        mesh=mesh,
        compiler_params=pltpu.CompilerParams(use_tc_tiling_on_sc=False),
        out_type=jax.ShapeDtypeStruct((NT, B, 128), jnp.bfloat16),
        scratch_types=[
            pltpu.VMEM((SH, BW), jnp.int32),
            pltpu.VMEM((BW, D), jnp.bfloat16),
            pltpu.VMEM((BW, D), jnp.bfloat16),
            pltpu.SemaphoreType.DMA,
            pltpu.SemaphoreType.DMA,
            pltpu.SemaphoreType.DMA,
            pltpu.SemaphoreType.DMA,
        ],
    )
    def sc_gather(table_hbm, idx_hbm, out_hbm, idx_v, rows0, rows1,
                  gs0, gs1, ws0, ws1):
        wid = lax.axis_index("s") * NC + lax.axis_index("c")
        bslice = wid % NB
        jhalf = wid // NB
        b0 = bslice * BW
        j_base = jhalf * SH
        pltpu.sync_copy(idx_hbm.at[pl.ds(j_base, SH), pl.ds(b0, BW)], idx_v)

        rows, gs, ws = [rows0, rows1], [gs0, gs1], [ws0, ws1]

        def wb_slice(j):
            jg = j_base + j
            return out_hbm.at[jg // DPT, pl.ds(b0, BW),
                              pl.ds((jg % DPT) * D, D)]

        def start_gather(j, k):
            return pltpu.make_async_copy(
                table_hbm.at[idx_v.at[j]], rows[k], gs[k])

        def body(jj, carry):
            j0 = jj * 2
            for k in range(2):
                @pl.when(jj >= 1)
                def _():
                    pltpu.make_async_copy(
                        rows[k], wb_slice(j0 + k), ws[k]).wait()
                start_gather(j0 + k, k).start()
            for k in range(2):
                pltpu.make_async_copy(
                    table_hbm.at[idx_v.at[j0 + k]], rows[k], gs[k]).wait()
                pltpu.make_async_copy(rows[k], wb_slice(j0 + k), ws[k]).start()
            return carry

        lax.fori_loop(0, SH // 2, body, 0)
        for k in range(2):
            pltpu.make_async_copy(rows[k], wb_slice(SH - 2 + k), ws[k]).wait()

    return sc_gather


@functools.cache
def _make_tc_mlp(B, NT, K, N1, N2, BM):
    def body(h_ref, w1_ref, b1_ref, w2_ref, b2_ref, o_ref):
        tail = K - (NT - 1) * 128
        acc = jnp.dot(h_ref[0], w1_ref[0],
                      preferred_element_type=jnp.float32)
        for t in range(1, NT - 1):
            acc += jnp.dot(h_ref[t], w1_ref[t],
                           preferred_element_type=jnp.float32)
        acc += jnp.dot(h_ref[NT - 1][:, :tail], w1_ref[NT - 1][:tail],
                       preferred_element_type=jnp.float32)
        z = jnp.maximum(acc + b1_ref[...], 0.0)
        logits = jnp.dot(z, w2_ref[...],
                         preferred_element_type=jnp.float32) + b2_ref[...]
        m = jnp.max(logits, axis=-1, keepdims=True)
        e = jnp.exp(logits - m)
        o_ref[...] = e / jnp.sum(e, axis=-1, keepdims=True)

    return pl.pallas_call(
        body,
        grid=(B // BM,),
        in_specs=[
            pl.BlockSpec((NT, BM, 128), lambda i: (0, i, 0)),
            pl.BlockSpec((NT, 128, N1), lambda i: (0, 0, 0)),
            pl.BlockSpec((1, N1), lambda i: (0, 0)),
            pl.BlockSpec((N1, N2), lambda i: (0, 0)),
            pl.BlockSpec((1, N2), lambda i: (0, 0)),
        ],
        out_specs=pl.BlockSpec((BM, N2), lambda i: (i, 0)),
        out_shape=jax.ShapeDtypeStruct((B, N2), jnp.float32),
    )


def kernel(x, emb, W1, b1, W2, b2):
    Bx, S = x.shape
    V, D = emb.shape
    K = S * D
    N1 = W1.shape[1]
    N2 = W2.shape[1]
    NT = (K + 127) // 128

    idx_t = x.T.astype(jnp.int32)
    emb_bf = emb.astype(jnp.bfloat16)
    w1p = jnp.zeros((NT * 128, N1), jnp.float32).at[:K].set(W1)
    w1p = w1p.reshape(NT, 128, N1).astype(jnp.bfloat16)
    b1r, b2r = b1.reshape(1, N1), b2.reshape(1, N2)

    P = 2
    BP = Bx // P
    outs = []
    for p in range(P):
        h3 = _make_sc_gather(V, D, S, BP, NT)(
            emb_bf, lax.dynamic_slice_in_dim(idx_t, p * BP, BP, axis=1))
        outs.append(_make_tc_mlp(BP, NT, K, N1, N2, 1024)(
            h3, w1p, b1r, W2, b2r))
    return jnp.concatenate(outs, axis=0)


# speedup vs baseline: 1.4508x; 1.4508x over previous
import functools

import jax
import jax.numpy as jnp
from jax import lax
from jax.experimental import pallas as pl
from jax.experimental.pallas import tpu as pltpu
from jax.experimental.pallas import tpu_sc as plsc


@functools.cache
def _make_sc_gather(V, D, S, B, NT):
    DW = D // 2
    M = B // 2
    info = plsc.get_sparse_core_info()
    NC, NS = info.num_cores, info.num_subcores
    NW = NC * NS
    NB = NW // 2
    assert M % NB == 0 and S % 2 == 0
    MW = M // NB
    SH = S // 2
    WPT = 64 // DW
    mesh = plsc.VectorSubcoreMesh(core_axis_name="c", subcore_axis_name="s")

    @functools.partial(
        pl.kernel,
        mesh=mesh,
        compiler_params=pltpu.CompilerParams(use_tc_tiling_on_sc=False),
        out_type=jax.ShapeDtypeStruct((NT, M, 128), jnp.int32),
        scratch_types=[
            pltpu.VMEM((SH, 2, MW), jnp.int32),
            pltpu.VMEM((MW, DW), jnp.int32),
            pltpu.VMEM((MW, DW), jnp.int32),
            pltpu.VMEM((MW, DW), jnp.int32),
            pltpu.VMEM((MW, DW), jnp.int32),
            pltpu.SemaphoreType.DMA,
            pltpu.SemaphoreType.DMA,
            pltpu.SemaphoreType.DMA,
            pltpu.SemaphoreType.DMA,
            pltpu.SemaphoreType.DMA,
            pltpu.SemaphoreType.DMA,
            pltpu.SemaphoreType.DMA,
            pltpu.SemaphoreType.DMA,
        ],
    )
    def sc_gather(table_hbm, idx_hbm, out_hbm, idx_v,
                  r00, r01, r10, r11,
                  g00, g01, g10, g11, w00, w01, w10, w11):
        wid = lax.axis_index("s") * NC + lax.axis_index("c")
        bslice = wid % NB
        jhalf = wid // NB
        m0 = bslice * MW
        j_base = jhalf * SH
        pltpu.sync_copy(
            idx_hbm.at[pl.ds(j_base, SH), :, pl.ds(m0, MW)], idx_v)

        rows = [[r00, r01], [r10, r11]]
        gs = [[g00, g01], [g10, g11]]
        ws = [[w00, w01], [w10, w11]]

        def wb_slice(j, e):
            jg = j_base + j
            col = e * 64 + (jg % WPT) * DW
            return out_hbm.at[jg // WPT, pl.ds(m0, MW), pl.ds(col, DW)]

        def body(jj, carry):
            j0 = jj * 2
            for k in range(2):
                for e in range(2):
                    @pl.when(jj >= 1)
                    def _():
                        pltpu.make_async_copy(
                            rows[k][e], wb_slice(j0 + k, e), ws[k][e]).wait()
                    pltpu.make_async_copy(
                        table_hbm.at[idx_v.at[j0 + k, e]],
                        rows[k][e], gs[k][e]).start()
            for k in range(2):
                for e in range(2):
                    pltpu.make_async_copy(
                        table_hbm.at[idx_v.at[j0 + k, e]],
                        rows[k][e], gs[k][e]).wait()
                    pltpu.make_async_copy(
                        rows[k][e], wb_slice(j0 + k, e), ws[k][e]).start()
            return carry

        lax.fori_loop(0, SH // 2, body, 0)
        for k in range(2):
            for e in range(2):
                pltpu.make_async_copy(
                    rows[k][e], wb_slice(SH - 2 + k, e), ws[k][e]).wait()

    return sc_gather


@functools.cache
def _make_tc_mlp(M, NT, K, N1, N2, BM):
    def half_mlp(h_ref, wlo_ref, whi_ref, b1_ref, w2_ref, b2_ref, e):
        tail = (K - (NT - 1) * 128) // 2

        def slab(t, width=64):
            return pltpu.bitcast(
                h_ref[t][:, e * 64:e * 64 + width], jnp.bfloat16)

        def acc_with(w_ref):
            a = jnp.dot(slab(0), w_ref[0], preferred_element_type=jnp.float32)
            for t in range(1, NT - 1):
                a += jnp.dot(slab(t), w_ref[t],
                             preferred_element_type=jnp.float32)
            a += jnp.dot(slab(NT - 1, tail), w_ref[NT - 1][:tail],
                         preferred_element_type=jnp.float32)
            return a

        acc_lo = acc_with(wlo_ref)
        acc_hi = acc_with(whi_ref)
        acc = acc_lo + jnp.concatenate(
            [acc_hi[1:], jnp.zeros_like(acc_hi[:1])], axis=0)
        z = jnp.maximum(acc + b1_ref[...], 0.0)
        logits = jnp.dot(z, w2_ref[...],
                         preferred_element_type=jnp.float32) + b2_ref[...]
        mx = jnp.max(logits, axis=-1, keepdims=True)
        ex = jnp.exp(logits - mx)
        return ex / jnp.sum(ex, axis=-1, keepdims=True)

    def body(h_ref, wlo_ref, whi_ref, b1_ref, w2_ref, b2_ref, o_ref):
        oe = half_mlp(h_ref, wlo_ref, whi_ref, b1_ref, w2_ref, b2_ref, 0)
        oo = half_mlp(h_ref, wlo_ref, whi_ref, b1_ref, w2_ref, b2_ref, 1)
        o_ref[...] = jnp.concatenate([oe, oo], axis=1)

    return pl.pallas_call(
        body,
        grid=(M // BM,),
        in_specs=[
            pl.BlockSpec((NT, BM, 128), lambda i: (0, i, 0)),
            pl.BlockSpec((NT, 64, N1), lambda i: (0, 0, 0)),
            pl.BlockSpec((NT, 64, N1), lambda i: (0, 0, 0)),
            pl.BlockSpec((1, N1), lambda i: (0, 0)),
            pl.BlockSpec((N1, N2), lambda i: (0, 0)),
            pl.BlockSpec((1, N2), lambda i: (0, 0)),
        ],
        out_specs=pl.BlockSpec((2 * BM, 2 * N2), lambda i: (i, 0)),
        out_shape=jax.ShapeDtypeStruct((2 * M, 2 * N2), jnp.float32),
    )


def kernel(x, emb, W1, b1, W2, b2):
    Bx, S = x.shape
    V, D = emb.shape
    K = S * D
    N1 = W1.shape[1]
    N2 = W2.shape[1]
    NT = (K + 127) // 128

    emb_i32 = lax.bitcast_convert_type(
        emb.astype(jnp.bfloat16).reshape(V, D // 2, 2), jnp.int32)
    idx_p = jnp.transpose(
        x.astype(jnp.int32).reshape(Bx // 2, 2, S), (2, 1, 0))
    w1p = jnp.zeros((NT * 128, N1), jnp.float32).at[:K].set(W1)
    w1p = w1p.reshape(NT, 128, N1).astype(jnp.bfloat16)
    w1lo = w1p[:, 0::2, :]
    w1hi = w1p[:, 1::2, :]
    b1r, b2r = b1.reshape(1, N1), b2.reshape(1, N2)

    P = 2
    BP = Bx // P
    outs = []
    for p in range(P):
        h3 = _make_sc_gather(V, D, S, BP, NT)(
            emb_i32,
            lax.dynamic_slice_in_dim(idx_p, p * BP // 2, BP // 2, axis=2))
        o2 = _make_tc_mlp(BP // 2, NT, K, N1, N2, 512)(
            h3, w1lo, w1hi, b1r, W2, b2r)
        outs.append(o2.reshape(BP // 2, 2, 2 * N2)[:, 0, :])
    return jnp.concatenate(outs, axis=0).reshape(Bx, N2)


# speedup vs baseline: 1.4515x; 1.0005x over previous
import functools

import jax
import jax.numpy as jnp
from jax import lax
from jax.experimental import pallas as pl
from jax.experimental.pallas import tpu as pltpu
from jax.experimental.pallas import tpu_sc as plsc


@functools.cache
def _make_sc_gather(V, D, S, B, NT):
    DW = D // 2
    M = B // 2
    info = plsc.get_sparse_core_info()
    NC, NS = info.num_cores, info.num_subcores
    NW = NC * NS
    NB = NW // 2
    assert M % NB == 0 and S % 2 == 0
    MW = M // NB
    SH = S // 2
    WPT = 64 // DW
    mesh = plsc.VectorSubcoreMesh(core_axis_name="c", subcore_axis_name="s")

    @functools.partial(
        pl.kernel,
        mesh=mesh,
        compiler_params=pltpu.CompilerParams(use_tc_tiling_on_sc=False),
        out_type=jax.ShapeDtypeStruct((NT, M, 128), jnp.int32),
        scratch_types=[
            pltpu.VMEM((SH, 2, MW), jnp.int32),
            pltpu.VMEM((MW, DW), jnp.int32),
            pltpu.VMEM((MW, DW), jnp.int32),
            pltpu.VMEM((MW, DW), jnp.int32),
            pltpu.VMEM((MW, DW), jnp.int32),
            pltpu.SemaphoreType.DMA,
            pltpu.SemaphoreType.DMA,
            pltpu.SemaphoreType.DMA,
            pltpu.SemaphoreType.DMA,
            pltpu.SemaphoreType.DMA,
            pltpu.SemaphoreType.DMA,
            pltpu.SemaphoreType.DMA,
            pltpu.SemaphoreType.DMA,
        ],
    )
    def sc_gather(table_hbm, idx_hbm, out_hbm, idx_v,
                  r00, r01, r10, r11,
                  g00, g01, g10, g11, w00, w01, w10, w11):
        wid = lax.axis_index("s") * NC + lax.axis_index("c")
        bslice = wid % NB
        jhalf = wid // NB
        m0 = bslice * MW
        j_base = jhalf * SH
        pltpu.sync_copy(
            idx_hbm.at[pl.ds(j_base, SH), :, pl.ds(m0, MW)], idx_v)

        rows = [[r00, r01], [r10, r11]]
        gs = [[g00, g01], [g10, g11]]
        ws = [[w00, w01], [w10, w11]]

        def wb_slice(j, e):
            jg = j_base + j
            col = e * 64 + (jg % WPT) * DW
            return out_hbm.at[jg // WPT, pl.ds(m0, MW), pl.ds(col, DW)]

        def body(jj, carry):
            j0 = jj * 2
            for k in range(2):
                for e in range(2):
                    @pl.when(jj >= 1)
                    def _():
                        pltpu.make_async_copy(
                            rows[k][e], wb_slice(j0 + k, e), ws[k][e]).wait()
                    pltpu.make_async_copy(
                        table_hbm.at[idx_v.at[j0 + k, e]],
                        rows[k][e], gs[k][e]).start()
            for k in range(2):
                for e in range(2):
                    pltpu.make_async_copy(
                        table_hbm.at[idx_v.at[j0 + k, e]],
                        rows[k][e], gs[k][e]).wait()
                    pltpu.make_async_copy(
                        rows[k][e], wb_slice(j0 + k, e), ws[k][e]).start()
            return carry

        lax.fori_loop(0, SH // 2, body, 0)
        for k in range(2):
            for e in range(2):
                pltpu.make_async_copy(
                    rows[k][e], wb_slice(SH - 2 + k, e), ws[k][e]).wait()

    return sc_gather


@functools.cache
def _make_tc_mlp(M, NT, K, N1, N2, BM):
    def half_mlp(h_ref, wcat_ref, b1_ref, w2_ref, b2_ref, e):
        tail = (K - (NT - 1) * 128) // 2

        def slab(t, width=64):
            return pltpu.bitcast(
                h_ref[t][:, e * 64:e * 64 + width], jnp.bfloat16)

        acc2 = jnp.dot(slab(0), wcat_ref[0],
                       preferred_element_type=jnp.float32)
        for t in range(1, NT - 1):
            acc2 += jnp.dot(slab(t), wcat_ref[t],
                            preferred_element_type=jnp.float32)
        acc2 += jnp.dot(slab(NT - 1, tail), wcat_ref[NT - 1][:tail],
                        preferred_element_type=jnp.float32)
        N1 = acc2.shape[1] // 2
        acc_lo = acc2[:, :N1]
        acc_hi = acc2[:, N1:]
        acc = acc_lo + jnp.concatenate(
            [acc_hi[1:], jnp.zeros_like(acc_hi[:1])], axis=0)
        z = jnp.maximum(acc + b1_ref[...], 0.0)
        logits = jnp.dot(z, w2_ref[...],
                         preferred_element_type=jnp.float32) + b2_ref[...]
        mx = jnp.max(logits, axis=-1, keepdims=True)
        ex = jnp.exp(logits - mx)
        return ex / jnp.sum(ex, axis=-1, keepdims=True)

    def body(h_ref, wcat_ref, b1_ref, w2_ref, b2_ref, o_ref):
        oe = half_mlp(h_ref, wcat_ref, b1_ref, w2_ref, b2_ref, 0)
        oo = half_mlp(h_ref, wcat_ref, b1_ref, w2_ref, b2_ref, 1)
        o_ref[...] = jnp.concatenate([oe, oo], axis=1)

    return pl.pallas_call(
        body,
        grid=(M // BM,),
        in_specs=[
            pl.BlockSpec((NT, BM, 128), lambda i: (0, i, 0)),
            pl.BlockSpec((NT, 64, 2 * N1), lambda i: (0, 0, 0)),
            pl.BlockSpec((1, N1), lambda i: (0, 0)),
            pl.BlockSpec((N1, N2), lambda i: (0, 0)),
            pl.BlockSpec((1, N2), lambda i: (0, 0)),
        ],
        out_specs=pl.BlockSpec((2 * BM, 2 * N2), lambda i: (i, 0)),
        out_shape=jax.ShapeDtypeStruct((2 * M, 2 * N2), jnp.float32),
    )


def kernel(x, emb, W1, b1, W2, b2):
    Bx, S = x.shape
    V, D = emb.shape
    K = S * D
    N1 = W1.shape[1]
    N2 = W2.shape[1]
    NT = (K + 127) // 128

    emb_i32 = lax.bitcast_convert_type(
        emb.astype(jnp.bfloat16).reshape(V, D // 2, 2), jnp.int32)
    idx_p = jnp.transpose(
        x.astype(jnp.int32).reshape(Bx // 2, 2, S), (2, 1, 0))
    w1p = jnp.zeros((NT * 128, N1), jnp.float32).at[:K].set(W1)
    w1p = w1p.reshape(NT, 128, N1).astype(jnp.bfloat16)
    w1cat = jnp.concatenate([w1p[:, 0::2, :], w1p[:, 1::2, :]], axis=2)
    b1r, b2r = b1.reshape(1, N1), b2.reshape(1, N2)

    P = 2
    BP = Bx // P
    outs = []
    for p in range(P):
        h3 = _make_sc_gather(V, D, S, BP, NT)(
            emb_i32,
            lax.dynamic_slice_in_dim(idx_p, p * BP // 2, BP // 2, axis=2))
        o2 = _make_tc_mlp(BP // 2, NT, K, N1, N2, 512)(
            h3, w1cat, b1r, W2, b2r)
        outs.append(o2.reshape(BP // 2, 2, 2 * N2)[:, 0, :])
    return jnp.concatenate(outs, axis=0).reshape(Bx, N2)
